# baseline (speedup 1.0000x reference)
import functools
import jax
import jax.numpy as jnp
from jax import lax
from jax.experimental import pallas as pl
from jax.experimental.pallas import tpu as pltpu
from jax.experimental.pallas import tpu_sc as plsc

N = 10000
D_IN = 128
HID = 128
H1 = 8
E = 320000

NC = 2
NS = 16
NW = NC * NS
NP = 10240
SL = NP // NS
E_PAD = 327680
EPW = E_PAD // NW
K = 128
CH = EPW // K
CH_A = 96
CH_B = 64
PAD_NODE = N

@functools.lru_cache(maxsize=None)
def _mesh():
    return plsc.VectorSubcoreMesh(
        core_axis_name="c", subcore_axis_name="s",
        num_cores=NC, num_subcores=NS)


def _mm1_body(x_ref, w_ref, ac_ref, h_ref, a_ref):
    h = jnp.dot(x_ref[...], w_ref[...], preferred_element_type=jnp.float32)
    h_ref[...] = h
    a_ref[...] = jnp.dot(h, ac_ref[...], preferred_element_type=jnp.float32)


def _mm1(x, W1, Acat):
    R = 400
    return pl.pallas_call(
        _mm1_body,
        grid=(N // R,),
        in_specs=[
            pl.BlockSpec((R, D_IN), lambda i: (i, 0)),
            pl.BlockSpec((D_IN, H1 * HID), lambda i: (0, 0)),
            pl.BlockSpec((H1 * HID, 16), lambda i: (0, 0)),
        ],
        out_specs=[
            pl.BlockSpec((R, H1 * HID), lambda i: (i, 0)),
            pl.BlockSpec((R, 16), lambda i: (i, 0)),
        ],
        out_shape=[
            jax.ShapeDtypeStruct((N, H1 * HID), jnp.float32),
            jax.ShapeDtypeStruct((N, 16), jnp.float32),
        ],
    )(x, W1, Acat)


def _mm2_body(p_ref, d_ref, b1_ref, w2_ref, a2c_ref, h2_ref, a2_ref):
    acc = jnp.zeros((512, HID), dtype=jnp.float32)
    for h in range(H1):
        inv = 1.0 / (d_ref[h, 0] + d_ref[h, 1] + 1e-16)
        pre = (p_ref[0, h] + p_ref[1, h]) * inv[:, None] + b1_ref[h][None, :]
        g = jnp.where(pre > 0, pre, jnp.exp(pre) - 1.0)
        acc = acc + jnp.dot(g, w2_ref[h], preferred_element_type=jnp.float32)
    h2_ref[...] = acc
    a2_ref[...] = jnp.dot(acc, a2c_ref[...], preferred_element_type=jnp.float32)


def _mm2(opart1, dpart1, b1r, W2r, A2cat):
    R = 512
    return pl.pallas_call(
        _mm2_body,
        grid=(NP // R,),
        in_specs=[
            pl.BlockSpec((2, H1, R, HID), lambda i: (0, 0, i, 0)),
            pl.BlockSpec((H1, 2, R), lambda i: (0, 0, i)),
            pl.BlockSpec((H1, HID), lambda i: (0, 0)),
            pl.BlockSpec((H1, HID, HID), lambda i: (0, 0, 0)),
            pl.BlockSpec((HID, 8), lambda i: (0, 0)),
        ],
        out_specs=[
            pl.BlockSpec((R, HID), lambda i: (i, 0)),
            pl.BlockSpec((R, 8), lambda i: (i, 0)),
        ],
        out_shape=[
            jax.ShapeDtypeStruct((NP, HID), jnp.float32),
            jax.ShapeDtypeStruct((NP, 8), jnp.float32),
        ],
    )(opart1, dpart1, b1r, W2r, A2cat)


def _fin_body(p_ref, d_ref, b2_ref, o_ref):
    inv = 1.0 / (d_ref[0] + d_ref[1] + 1e-16)
    o_ref[...] = (p_ref[0] + p_ref[1]) * inv[:, None] + b2_ref[...]


def _fin(opart2, dpart2f, b2r):
    R = 1024
    return pl.pallas_call(
        _fin_body,
        grid=(NP // R,),
        in_specs=[
            pl.BlockSpec((2, R, HID), lambda i: (0, i, 0)),
            pl.BlockSpec((2, R), lambda i: (0, i)),
            pl.BlockSpec((1, HID), lambda i: (0, 0)),
        ],
        out_specs=pl.BlockSpec((R, HID), lambda i: (i, 0)),
        out_shape=jax.ShapeDtypeStruct((NP, HID), jnp.float32),
    )(opart2, dpart2f, b2r)


def _make_att(H):
    def body(aT, src1d, dst1d, ex, dpart,
             asrc_v, adst_v, dacc_v, srcv, dstv, exv, red_v, acc_v, dsh):
        cid = lax.axis_index("c")
        sid = lax.axis_index("s")
        wid = cid * NS + sid
        ebase = wid * EPW
        pltpu.sync_copy(src1d.at[pl.ds(ebase, EPW)], srcv)
        pltpu.sync_copy(dst1d.at[pl.ds(ebase, EPW)], dstv)
        off = sid * SL
        for h in range(H):
            pltpu.sync_copy(aT.at[h], asrc_v)
            pltpu.sync_copy(aT.at[H + h], adst_v)

            @pl.loop(0, NP // 16)
            def _zero(i):
                dacc_v[pl.ds(i * 16, 16)] = jnp.zeros((16,), jnp.float32)

            @pl.loop(0, EPW // 16)
            def _edges(i):
                b = i * 16
                s16 = srcv[pl.ds(b, 16)]
                d16 = dstv[pl.ds(b, 16)]
                sa = plsc.load_gather(asrc_v, [s16])
                da = plsc.load_gather(adst_v, [d16])
                al = sa + da
                al = jnp.maximum(al, al * 0.2)
                e16 = jnp.exp(al)
                exv[pl.ds(b, 16)] = e16
                plsc.addupdate_scatter(dacc_v, [d16], e16)

            pltpu.sync_copy(exv, ex.at[h, pl.ds(ebase, EPW)])
            pltpu.sync_copy(dacc_v, dsh.at[sid])
            plsc.subcore_barrier()
            pltpu.sync_copy(dsh.at[:, pl.ds(off, SL)], red_v)

            @pl.loop(0, SL // 16)
            def _red(j):
                t = red_v[0, pl.ds(j * 16, 16)]
                for r in range(1, NS):
                    t = t + red_v[r, pl.ds(j * 16, 16)]
                acc_v[pl.ds(j * 16, 16)] = t

            pltpu.sync_copy(acc_v, dpart.at[h, cid, pl.ds(off, SL)])
            plsc.subcore_barrier()

    RA = 2 * H if H > 1 else 8

    @functools.partial(
        pl.kernel,
        out_type=[
            jax.ShapeDtypeStruct((H, E_PAD), jnp.float32),
            jax.ShapeDtypeStruct((H, NC, NP), jnp.float32),
        ],
        mesh=_mesh(),
        scratch_types=[
            pltpu.VMEM((NP,), jnp.float32),
            pltpu.VMEM((NP,), jnp.float32),
            pltpu.VMEM((NP,), jnp.float32),
            pltpu.VMEM((EPW,), jnp.int32),
            pltpu.VMEM((EPW,), jnp.int32),
            pltpu.VMEM((EPW,), jnp.float32),
            pltpu.VMEM((NS, SL), jnp.float32),
            pltpu.VMEM((SL,), jnp.float32),
            pltpu.VMEM_SHARED((NS, NP), jnp.float32),
        ],
        compiler_params=pltpu.CompilerParams(needs_layout_passes=False),
        name=f"gat_att_h{H}",
    )
    def kern(aT, src1d, dst1d, ex, dpart, *scratch):
        body(aT, src1d, dst1d, ex, dpart, *scratch)

    return kern


def _make_agg(H):
    def body(rows_hbm, src1d, dst1d, ex, opart,
             srck, exk, didx, idxb, rows, zb_v, acc_sh, sems, ssem):
        cid = lax.axis_index("c")
        sid = lax.axis_index("s")
        nch = jnp.where(cid == 0, CH_A, CH_B)
        ebase = jnp.where(cid == 0, sid * (CH_A * K),
                          NS * (CH_A * K) + sid * (CH_B * K))
        off = sid * SL

        @pl.loop(0, 32)
        def _zb(i):
            r = zb_v.at[i]
            for j in range(8):
                r[pl.ds(j * 16, 16)] = jnp.zeros((16,), jnp.float32)

        @pl.loop(0, H)
        def _head(h):
            @pl.loop(0, SL // 32)
            def _zacc(i):
                pltpu.sync_copy(zb_v, acc_sh.at[pl.ds(off + i * 32, 32)])

            plsc.subcore_barrier()

            def stage(ch, b):
                eb = ebase + ch * K
                pltpu.sync_copy(src1d.at[pl.ds(eb, K)], srck[b])
                pltpu.sync_copy(dst1d.at[pl.ds(eb, K)], didx[b])
                pltpu.sync_copy(ex.at[h, pl.ds(eb, K)], exk[b])
                if H == 1:
                    pltpu.async_copy(rows_hbm.at[srck[b]], rows[b], sems[b])
                else:
                    @pl.loop(0, K // 16)
                    def _idx(j):
                        s16 = srck[b][pl.ds(j * 16, 16)]
                        idxb[b][pl.ds(j * 16, 16)] = s16 * H + h
                    pltpu.async_copy(rows_hbm.at[idxb[b]], rows[b], sems[b])

            def consume(b):
                iref = srck[b] if H == 1 else idxb[b]
                pltpu.make_async_copy(rows_hbm.at[iref], rows[b],
                                      sems[b]).wait()

                @pl.loop(0, K, unroll=8)
                def _scale(i):
                    c16 = plsc.load_gather(
                        exk[b], [jnp.full((16,), i, jnp.int32)])
                    r = rows[b].at[i]
                    for j in range(8):
                        sl = pl.ds(j * 16, 16)
                        r[sl] = r[sl] * c16

                pltpu.sync_copy(rows[b], acc_sh.at[didx[b]], add=True)

            stage(0, 0)
            stage(1, 1)

            @pl.loop(0, nch // 2 - 1)
            def _group(g):
                for b in range(2):
                    consume(b)
                    stage(g * 2 + b + 2, b)

            for b in range(2):
                consume(b)

            plsc.subcore_barrier()
            pltpu.sync_copy(acc_sh.at[pl.ds(off, SL)],
                            opart.at[cid, h, pl.ds(off, SL)])
            plsc.subcore_barrier()

    @functools.partial(
        pl.kernel,
        out_type=jax.ShapeDtypeStruct((NC, H, NP, HID), jnp.float32),
        mesh=_mesh(),
        scratch_types=[
            [pltpu.VMEM((K,), jnp.int32)] * 2,
            [pltpu.VMEM((K,), jnp.float32)] * 2,
            [pltpu.VMEM((K,), jnp.int32)] * 2,
            [pltpu.VMEM((K,), jnp.int32)] * 2,
            [pltpu.VMEM((K, HID), jnp.float32)] * 2,
            pltpu.VMEM((32, HID), jnp.float32),
            pltpu.VMEM_SHARED((NP, HID), jnp.float32),
            [pltpu.SemaphoreType.DMA] * 2,
            [pltpu.SemaphoreType.DMA] * 2,
        ],
        compiler_params=pltpu.CompilerParams(needs_layout_passes=False),
        name=f"gat_agg_h{H}",
    )
    def kern(rows_hbm, src1d, dst1d, ex, opart, *scratch):
        body(rows_hbm, src1d, dst1d, ex, opart, *scratch)

    return kern


@functools.lru_cache(maxsize=None)
def _sc_kernels():
    return _make_att(H1), _make_att(1), _make_agg(H1), _make_agg(1)


@jax.jit
def kernel(x, edge_list, W1, a_src1, a_dst1, b1, W2, a_src2, a_dst2, b2):
    f32 = jnp.float32
    src = edge_list[0].astype(jnp.int32)
    dst = edge_list[1].astype(jnp.int32)
    src_p = jnp.concatenate(
        [src, jnp.full((E_PAD - E,), PAD_NODE, jnp.int32)])
    dst_p = jnp.concatenate(
        [dst, jnp.full((E_PAD - E,), PAD_NODE, jnp.int32)])

    A = jnp.zeros((H1, HID, 16), f32)
    for h in range(H1):
        A = A.at[h, :, h].set(a_src1[h])
        A = A.at[h, :, 8 + h].set(a_dst1[h])
    Acat = A.reshape(H1 * HID, 16)

    h1, a1 = _mm1(x, W1, Acat)
    a1T = jnp.pad(a1.T, ((0, 0), (0, NP - N)))
    h1rows = jnp.pad(h1, ((0, 16), (0, 0))).reshape((N + 16) * H1, HID)

    _att1, _att2, _agg1, _agg2 = _sc_kernels()
    ex1, dpart1 = _att1(a1T, src_p, dst_p)
    opart1 = _agg1(h1rows, src_p, dst_p, ex1)

    b1r = b1.reshape(H1, HID)
    W2r = W2.reshape(H1, HID, HID)
    A2cat = jnp.zeros((HID, 8), f32)
    A2cat = A2cat.at[:, 0].set(a_src2[0]).at[:, 1].set(a_dst2[0])
    h2, a2 = _mm2(opart1, dpart1, b1r, W2r, A2cat)

    a2T = a2.T
    ex2, dpart2 = _att2(a2T, src_p, dst_p)
    opart2 = _agg2(h2, src_p, dst_p, ex2)

    out = _fin(opart2.reshape(NC, NP, HID), dpart2[0], b2.reshape(1, HID))
    return out[:N]


# speedup vs baseline: 1.0341x; 1.0341x over previous
import functools
import jax
import jax.numpy as jnp
from jax import lax
from jax.experimental import pallas as pl
from jax.experimental.pallas import tpu as pltpu
from jax.experimental.pallas import tpu_sc as plsc

N = 10000
D_IN = 128
HID = 128
H1 = 8
E = 320000

NC = 2
NS = 16
NW = NC * NS
NP = 10240
SL = NP // NS
E_PAD = 327680
EPW = E_PAD // NW
K = 128
CH = EPW // K
CH_A = 112
CH_B = 48
PAD_NODE = N

@functools.lru_cache(maxsize=None)
def _mesh():
    return plsc.VectorSubcoreMesh(
        core_axis_name="c", subcore_axis_name="s",
        num_cores=NC, num_subcores=NS)


def _mm1_body(x_ref, w_ref, ac_ref, h_ref, a_ref):
    h = jnp.dot(x_ref[...], w_ref[...], preferred_element_type=jnp.float32)
    h_ref[...] = h
    a_ref[...] = jnp.dot(h, ac_ref[...], preferred_element_type=jnp.float32)


def _mm1(x, W1, Acat):
    R = 400
    return pl.pallas_call(
        _mm1_body,
        grid=(N // R,),
        in_specs=[
            pl.BlockSpec((R, D_IN), lambda i: (i, 0)),
            pl.BlockSpec((D_IN, H1 * HID), lambda i: (0, 0)),
            pl.BlockSpec((H1 * HID, 16), lambda i: (0, 0)),
        ],
        out_specs=[
            pl.BlockSpec((R, H1 * HID), lambda i: (i, 0)),
            pl.BlockSpec((R, 16), lambda i: (i, 0)),
        ],
        out_shape=[
            jax.ShapeDtypeStruct((N, H1 * HID), jnp.float32),
            jax.ShapeDtypeStruct((N, 16), jnp.float32),
        ],
    )(x, W1, Acat)


def _mm2_body(p_ref, d_ref, b1_ref, w2_ref, a2c_ref, h2_ref, a2_ref):
    acc = jnp.zeros((512, HID), dtype=jnp.float32)
    for h in range(H1):
        inv = 1.0 / (d_ref[h, 0] + d_ref[h, 1] + 1e-16)
        pre = (p_ref[0, h] + p_ref[1, h]) * inv[:, None] + b1_ref[h][None, :]
        g = jnp.where(pre > 0, pre, jnp.exp(pre) - 1.0)
        acc = acc + jnp.dot(g, w2_ref[h], preferred_element_type=jnp.float32)
    h2_ref[...] = acc
    a2_ref[...] = jnp.dot(acc, a2c_ref[...], preferred_element_type=jnp.float32)


def _mm2(opart1, dpart1, b1r, W2r, A2cat):
    R = 512
    return pl.pallas_call(
        _mm2_body,
        grid=(NP // R,),
        in_specs=[
            pl.BlockSpec((2, H1, R, HID), lambda i: (0, 0, i, 0)),
            pl.BlockSpec((H1, 2, R), lambda i: (0, 0, i)),
            pl.BlockSpec((H1, HID), lambda i: (0, 0)),
            pl.BlockSpec((H1, HID, HID), lambda i: (0, 0, 0)),
            pl.BlockSpec((HID, 8), lambda i: (0, 0)),
        ],
        out_specs=[
            pl.BlockSpec((R, HID), lambda i: (i, 0)),
            pl.BlockSpec((R, 8), lambda i: (i, 0)),
        ],
        out_shape=[
            jax.ShapeDtypeStruct((NP, HID), jnp.float32),
            jax.ShapeDtypeStruct((NP, 8), jnp.float32),
        ],
    )(opart1, dpart1, b1r, W2r, A2cat)


def _fin_body(p_ref, d_ref, b2_ref, o_ref):
    inv = 1.0 / (d_ref[0] + d_ref[1] + 1e-16)
    o_ref[...] = (p_ref[0] + p_ref[1]) * inv[:, None] + b2_ref[...]


def _fin(opart2, dpart2f, b2r):
    R = 1024
    return pl.pallas_call(
        _fin_body,
        grid=(NP // R,),
        in_specs=[
            pl.BlockSpec((2, R, HID), lambda i: (0, i, 0)),
            pl.BlockSpec((2, R), lambda i: (0, i)),
            pl.BlockSpec((1, HID), lambda i: (0, 0)),
        ],
        out_specs=pl.BlockSpec((R, HID), lambda i: (i, 0)),
        out_shape=jax.ShapeDtypeStruct((NP, HID), jnp.float32),
    )(opart2, dpart2f, b2r)


def _make_att(H):
    def body(aT, src1d, dst1d, ex, dpart,
             asrc_v, adst_v, dacc_v, srcv, dstv, exv, red_v, acc_v, dsh):
        cid = lax.axis_index("c")
        sid = lax.axis_index("s")
        wid = cid * NS + sid
        ebase = wid * EPW
        pltpu.sync_copy(src1d.at[pl.ds(ebase, EPW)], srcv)
        pltpu.sync_copy(dst1d.at[pl.ds(ebase, EPW)], dstv)
        off = sid * SL
        for h in range(H):
            pltpu.sync_copy(aT.at[h], asrc_v)
            pltpu.sync_copy(aT.at[H + h], adst_v)

            @pl.loop(0, NP // 16)
            def _zero(i):
                dacc_v[pl.ds(i * 16, 16)] = jnp.zeros((16,), jnp.float32)

            @pl.loop(0, EPW // 16)
            def _edges(i):
                b = i * 16
                s16 = srcv[pl.ds(b, 16)]
                d16 = dstv[pl.ds(b, 16)]
                sa = plsc.load_gather(asrc_v, [s16])
                da = plsc.load_gather(adst_v, [d16])
                al = sa + da
                al = jnp.maximum(al, al * 0.2)
                e16 = jnp.exp(al)
                exv[pl.ds(b, 16)] = e16
                plsc.addupdate_scatter(dacc_v, [d16], e16)

            pltpu.sync_copy(exv, ex.at[h, pl.ds(ebase, EPW)])
            pltpu.sync_copy(dacc_v, dsh.at[sid])
            plsc.subcore_barrier()
            pltpu.sync_copy(dsh.at[:, pl.ds(off, SL)], red_v)

            @pl.loop(0, SL // 16)
            def _red(j):
                t = red_v[0, pl.ds(j * 16, 16)]
                for r in range(1, NS):
                    t = t + red_v[r, pl.ds(j * 16, 16)]
                acc_v[pl.ds(j * 16, 16)] = t

            pltpu.sync_copy(acc_v, dpart.at[h, cid, pl.ds(off, SL)])
            plsc.subcore_barrier()

    RA = 2 * H if H > 1 else 8

    @functools.partial(
        pl.kernel,
        out_type=[
            jax.ShapeDtypeStruct((H, E_PAD), jnp.float32),
            jax.ShapeDtypeStruct((H, NC, NP), jnp.float32),
        ],
        mesh=_mesh(),
        scratch_types=[
            pltpu.VMEM((NP,), jnp.float32),
            pltpu.VMEM((NP,), jnp.float32),
            pltpu.VMEM((NP,), jnp.float32),
            pltpu.VMEM((EPW,), jnp.int32),
            pltpu.VMEM((EPW,), jnp.int32),
            pltpu.VMEM((EPW,), jnp.float32),
            pltpu.VMEM((NS, SL), jnp.float32),
            pltpu.VMEM((SL,), jnp.float32),
            pltpu.VMEM_SHARED((NS, NP), jnp.float32),
        ],
        compiler_params=pltpu.CompilerParams(needs_layout_passes=False),
        name=f"gat_att_h{H}",
    )
    def kern(aT, src1d, dst1d, ex, dpart, *scratch):
        body(aT, src1d, dst1d, ex, dpart, *scratch)

    return kern


def _make_agg(H):
    def body(rows_hbm, src1d, dst1d, ex, opart,
             srck, exk, didx, idxb, rows, zb_v, acc_sh, sems, ssem):
        cid = lax.axis_index("c")
        sid = lax.axis_index("s")
        nch = jnp.where(cid == 0, CH_A, CH_B)
        ebase = jnp.where(cid == 0, sid * (CH_A * K),
                          NS * (CH_A * K) + sid * (CH_B * K))
        off = sid * SL

        @pl.loop(0, 32)
        def _zb(i):
            r = zb_v.at[i]
            for j in range(8):
                r[pl.ds(j * 16, 16)] = jnp.zeros((16,), jnp.float32)

        @pl.loop(0, H)
        def _head(h):
            @pl.loop(0, SL // 32)
            def _zacc(i):
                pltpu.sync_copy(zb_v, acc_sh.at[pl.ds(off + i * 32, 32)])

            plsc.subcore_barrier()

            def stage(ch, b):
                eb = ebase + ch * K
                pltpu.sync_copy(src1d.at[pl.ds(eb, K)], srck[b])
                pltpu.sync_copy(dst1d.at[pl.ds(eb, K)], didx[b])
                pltpu.sync_copy(ex.at[h, pl.ds(eb, K)], exk[b])
                if H == 1:
                    pltpu.async_copy(rows_hbm.at[srck[b]], rows[b], sems[b])
                else:
                    @pl.loop(0, K // 16)
                    def _idx(j):
                        s16 = srck[b][pl.ds(j * 16, 16)]
                        idxb[b][pl.ds(j * 16, 16)] = s16 * H + h
                    pltpu.async_copy(rows_hbm.at[idxb[b]], rows[b], sems[b])

            def consume(b):
                iref = srck[b] if H == 1 else idxb[b]
                pltpu.make_async_copy(rows_hbm.at[iref], rows[b],
                                      sems[b]).wait()

                @pl.loop(0, K, unroll=8)
                def _scale(i):
                    c16 = plsc.load_gather(
                        exk[b], [jnp.full((16,), i, jnp.int32)])
                    r = rows[b].at[i]
                    for j in range(8):
                        sl = pl.ds(j * 16, 16)
                        r[sl] = r[sl] * c16

                pltpu.sync_copy(rows[b], acc_sh.at[didx[b]], add=True)

            stage(0, 0)
            stage(1, 1)

            @pl.loop(0, nch // 2 - 1)
            def _group(g):
                for b in range(2):
                    consume(b)
                    stage(g * 2 + b + 2, b)

            for b in range(2):
                consume(b)

            plsc.subcore_barrier()
            pltpu.sync_copy(acc_sh.at[pl.ds(off, SL)],
                            opart.at[cid, h, pl.ds(off, SL)])
            plsc.subcore_barrier()

    @functools.partial(
        pl.kernel,
        out_type=jax.ShapeDtypeStruct((NC, H, NP, HID), jnp.float32),
        mesh=_mesh(),
        scratch_types=[
            [pltpu.VMEM((K,), jnp.int32)] * 2,
            [pltpu.VMEM((K,), jnp.float32)] * 2,
            [pltpu.VMEM((K,), jnp.int32)] * 2,
            [pltpu.VMEM((K,), jnp.int32)] * 2,
            [pltpu.VMEM((K, HID), jnp.float32)] * 2,
            pltpu.VMEM((32, HID), jnp.float32),
            pltpu.VMEM_SHARED((NP, HID), jnp.float32),
            [pltpu.SemaphoreType.DMA] * 2,
            [pltpu.SemaphoreType.DMA] * 2,
        ],
        compiler_params=pltpu.CompilerParams(needs_layout_passes=False),
        name=f"gat_agg_h{H}",
    )
    def kern(rows_hbm, src1d, dst1d, ex, opart, *scratch):
        body(rows_hbm, src1d, dst1d, ex, opart, *scratch)

    return kern


@functools.lru_cache(maxsize=None)
def _sc_kernels():
    return _make_att(H1), _make_att(1), _make_agg(H1), _make_agg(1)


@jax.jit
def kernel(x, edge_list, W1, a_src1, a_dst1, b1, W2, a_src2, a_dst2, b2):
    f32 = jnp.float32
    src = edge_list[0].astype(jnp.int32)
    dst = edge_list[1].astype(jnp.int32)
    src_p = jnp.concatenate(
        [src, jnp.full((E_PAD - E,), PAD_NODE, jnp.int32)])
    dst_p = jnp.concatenate(
        [dst, jnp.full((E_PAD - E,), PAD_NODE, jnp.int32)])

    A = jnp.zeros((H1, HID, 16), f32)
    for h in range(H1):
        A = A.at[h, :, h].set(a_src1[h])
        A = A.at[h, :, 8 + h].set(a_dst1[h])
    Acat = A.reshape(H1 * HID, 16)

    h1, a1 = _mm1(x, W1, Acat)
    a1T = jnp.pad(a1.T, ((0, 0), (0, NP - N)))
    h1rows = jnp.pad(h1, ((0, 16), (0, 0))).reshape((N + 16) * H1, HID)

    _att1, _att2, _agg1, _agg2 = _sc_kernels()
    ex1, dpart1 = _att1(a1T, src_p, dst_p)
    opart1 = _agg1(h1rows, src_p, dst_p, ex1)

    b1r = b1.reshape(H1, HID)
    W2r = W2.reshape(H1, HID, HID)
    A2cat = jnp.zeros((HID, 8), f32)
    A2cat = A2cat.at[:, 0].set(a_src2[0]).at[:, 1].set(a_dst2[0])
    h2, a2 = _mm2(opart1, dpart1, b1r, W2r, A2cat)

    a2T = a2.T
    ex2, dpart2 = _att2(a2T, src_p, dst_p)
    opart2 = _agg2(h2, src_p, dst_p, ex2)

    out = _fin(opart2.reshape(NC, NP, HID), dpart2[0], b2.reshape(1, HID))
    return out[:N]
